# baseline (speedup 1.0000x reference)
import functools

import jax
import jax.numpy as jnp
from jax import lax
from jax.experimental import pallas as pl
from jax.experimental.pallas import tpu as pltpu
from jax.experimental.pallas import tpu_sc as plsc

_N = 10000
_E = 320000
_G = 64
_NSUB = 16
_B = 128
_K = 8
_NRND = 20
_NRI = _NRND + 1
_EPS = _E // _NSUB
_NPAD = 10016
_RPS = 624
_RTAIL = _N - _NSUB * _RPS
_BN = 1000


def _make_spmm(nchunk, fc):
    cpc = nchunk // 2
    mesh = plsc.VectorSubcoreMesh(core_axis_name="c", subcore_axis_name="s")

    @functools.partial(
        pl.kernel,
        out_type=jax.ShapeDtypeStruct((nchunk * _N, fc), jnp.float32),
        mesh=mesh,
        scratch_types=[
            pltpu.VMEM((_NRI, _K * _B), jnp.int32),
            pltpu.VMEM((_NRI, _K * _B), jnp.int32),
            [pltpu.VMEM((_K * _B, fc), jnp.float32) for _ in range(2)],
            pltpu.VMEM_SHARED((_NPAD, fc), jnp.float32),
            pltpu.SemaphoreType.DMA((2,)),
        ],
        compiler_params=pltpu.CompilerParams(use_tc_tiling_on_sc=False),
    )
    def spmm(y_hbm, srcq_hbm, dst_hbm, out_hbm, src_v, dst_v, rows, acc, gsem):
        c = lax.axis_index("c")
        s = lax.axis_index("s")

        def g_issue(b, r):
            pltpu.async_copy(y_hbm.at[src_v.at[r]], rows[b], gsem.at[b])

        def g_wait(b, r):
            pltpu.make_async_copy(y_hbm.at[src_v.at[r]], rows[b],
                                  gsem.at[b]).wait()

        pltpu.sync_copy(dst_hbm.at[s], dst_v)
        for j in range(cpc):
            q = c * cpc + j
            pltpu.sync_copy(srcq_hbm.at[q, s], src_v)
            pltpu.sync_copy(y_hbm.at[pl.ds(q * _N + s * _RPS, _RPS)],
                            acc.at[pl.ds(s * _RPS, _RPS)])

            @pl.when(s == 0)
            def _():
                pltpu.sync_copy(
                    y_hbm.at[pl.ds(q * _N + _NSUB * _RPS, _RTAIL)],
                    acc.at[pl.ds(_NSUB * _RPS, _RTAIL)])

            plsc.subcore_barrier()

            g_issue(0, 0)

            def rbody2(rr, carry):
                r0 = rr * 2
                g_wait(0, r0)
                g_issue(1, r0 + 1)
                pltpu.sync_copy(rows[0], acc.at[dst_v.at[r0]], add=True)
                g_wait(1, r0 + 1)
                g_issue(0, r0 + 2)
                pltpu.sync_copy(rows[1], acc.at[dst_v.at[r0 + 1]], add=True)
                return carry

            lax.fori_loop(0, _NRND // 2, rbody2, 0)
            g_wait(0, _NRND)

            plsc.subcore_barrier()
            pltpu.sync_copy(acc.at[pl.ds(s * _RPS, _RPS)],
                            out_hbm.at[pl.ds(q * _N + s * _RPS, _RPS)])

            @pl.when(s == 0)
            def _():
                pltpu.sync_copy(
                    acc.at[pl.ds(_NSUB * _RPS, _RTAIL)],
                    out_hbm.at[pl.ds(q * _N + _NSUB * _RPS, _RTAIL)])

            if j + 1 < cpc:
                plsc.subcore_barrier()

    return spmm


def _l1_body(x_ref, deg_ref, w_ref, out_ref):
    dinv = lax.rsqrt(deg_ref[...])
    y = jnp.dot(x_ref[...] * dinv, w_ref[...],
                preferred_element_type=jnp.float32)
    for q in range(4):
        out_ref[q] = y[:, q * 32:(q + 1) * 32]


def _make_layer_body(nc_in, nc_out, fco):
    def body(a_ref, deg_ref, b_ref, w_ref, out_ref):
        dinv = lax.rsqrt(deg_ref[...])
        h = jnp.concatenate([a_ref[i] for i in range(nc_in)], axis=1)
        h = jax.nn.relu(h * dinv + b_ref[...])
        y = jnp.dot(h * dinv, w_ref[...], preferred_element_type=jnp.float32)
        for q in range(nc_out):
            out_ref[q] = y[:, q * fco:(q + 1) * fco]
    return body


def _pool_body(a_ref, deg_ref, b_ref, batch_ref, gs_ref, gmp_ref):
    i = pl.program_id(0)
    dinv = lax.rsqrt(deg_ref[...])
    h = jnp.concatenate([a_ref[q] for q in range(16)], axis=1)
    h = jax.nn.relu(h * dinv + b_ref[...])
    gid = lax.broadcasted_iota(jnp.int32, (1, _G), 1)
    onehot = (batch_ref[...] == gid).astype(jnp.float32)
    gs = lax.dot_general(onehot, h, (((0,), (0,)), ((), ())),
                         preferred_element_type=jnp.float32)
    parts = []
    for g in range(_G):
        parts.append(jnp.max(onehot[:, g:g + 1] * h, axis=0, keepdims=True))
    gmp = jnp.concatenate(parts, axis=0)

    @pl.when(i == 0)
    def _():
        gs_ref[...] = gs
        gmp_ref[...] = gmp

    @pl.when(i > 0)
    def _():
        gs_ref[...] += gs
        gmp_ref[...] = jnp.maximum(gmp_ref[...], gmp)


def _mlp_body(batch_ref, gs_ref, gmp_ref, sf_ref,
              Wg1_ref, bg1_ref, Wg2_ref, bg2_ref,
              Ws1_ref, bs1_ref, Ws2_ref, bs2_ref,
              Wf1_ref, bf1_ref, Wf2_ref, bf2_ref, Wo_ref, bo_ref, out_ref):
    gid = lax.broadcasted_iota(jnp.int32, (1, _G), 1)
    onehot = (batch_ref[...] == gid).astype(jnp.float32)
    ones = jnp.ones((_N, 1), jnp.float32)
    counts = lax.dot_general(onehot, ones, (((0,), (0,)), ((), ())),
                             preferred_element_type=jnp.float32)
    gap = gs_ref[...] / jnp.maximum(counts, 1.0)
    comb = jnp.concatenate([gap, gmp_ref[...]], axis=1)
    comb = jax.nn.relu(
        jnp.dot(comb, Wg1_ref[...], preferred_element_type=jnp.float32)
        + bg1_ref[...])
    comb = jax.nn.relu(
        jnp.dot(comb, Wg2_ref[...], preferred_element_type=jnp.float32)
        + bg2_ref[...])
    s = jax.nn.relu(
        jnp.dot(sf_ref[...], Ws1_ref[...], preferred_element_type=jnp.float32)
        + bs1_ref[...])
    s = jax.nn.relu(
        jnp.dot(s, Ws2_ref[...], preferred_element_type=jnp.float32)
        + bs2_ref[...])
    z = jnp.concatenate([comb, s], axis=1)
    z = jax.nn.relu(
        jnp.dot(z, Wf1_ref[...], preferred_element_type=jnp.float32)
        + bf1_ref[...])
    z = jax.nn.relu(
        jnp.dot(z, Wf2_ref[...], preferred_element_type=jnp.float32)
        + bf2_ref[...])
    out_ref[...] = (
        jnp.dot(z, Wo_ref[...], preferred_element_type=jnp.float32)
        + bo_ref[...])


def _layer_call(body, nc_in, fci, nc_out, fco, a, deg2, b, w):
    return pl.pallas_call(
        body,
        grid=(_N // _BN,),
        in_specs=[
            pl.BlockSpec((nc_in, _BN, fci), lambda i: (0, i, 0)),
            pl.BlockSpec((_BN, 1), lambda i: (i, 0)),
            pl.BlockSpec((1, nc_in * fci), lambda i: (0, 0)),
            pl.BlockSpec((nc_in * fci, nc_out * fco), lambda i: (0, 0)),
        ],
        out_specs=pl.BlockSpec((nc_out, _BN, fco), lambda i: (0, i, 0)),
        out_shape=jax.ShapeDtypeStruct((nc_out, _N, fco), jnp.float32),
    )(a, deg2, b, w)


def kernel(x, edge_index, edge_attr, batch, solvent_fingerprint,
           W1, b1, W2, b2, W3, b3, Wg1, bg1, Wg2, bg2,
           Ws1, bs1, Ws2, bs2, Wf1, bf1, Wf2, bf2, Wo, bo):
    src = edge_index[0]
    dst = edge_index[1]
    padw = _NRI * _K * _B - _EPS
    src16 = jnp.pad(src.reshape(_NSUB, _EPS), ((0, 0), (0, padw)))
    qoff = (jnp.arange(16, dtype=jnp.int32) * _N)[:, None, None, None]
    srcq = src16.reshape(1, _NSUB, _NRI, _K * _B) + qoff
    dst_p = jnp.pad(dst.reshape(_NSUB, _EPS), ((0, 0), (0, padw)),
                    constant_values=_N).reshape(_NSUB, _NRI, _K * _B)

    spmm16 = _make_spmm(2, 16)
    deg_full = spmm16(jnp.ones((2 * _N, 16), jnp.float32), srcq, dst_p)
    deg2 = deg_full[:_N, :1]

    y1 = pl.pallas_call(
        _l1_body,
        grid=(_N // _BN,),
        in_specs=[
            pl.BlockSpec((_BN, 128), lambda i: (i, 0)),
            pl.BlockSpec((_BN, 1), lambda i: (i, 0)),
            pl.BlockSpec((128, 128), lambda i: (0, 0)),
        ],
        out_specs=pl.BlockSpec((4, _BN, 32), lambda i: (0, i, 0)),
        out_shape=jax.ShapeDtypeStruct((4, _N, 32), jnp.float32),
    )(x, deg2, W1)
    spmm32 = _make_spmm(4, 32)
    agg1 = spmm32(y1.reshape(4 * _N, 32), srcq, dst_p)

    y2 = _layer_call(_make_layer_body(4, 8, 32), 4, 32, 8, 32,
                     agg1.reshape(4, _N, 32), deg2, b1.reshape(1, 128), W2)
    spmm32x8 = _make_spmm(8, 32)
    agg2 = spmm32x8(y2.reshape(8 * _N, 32), srcq, dst_p)

    y3 = _layer_call(_make_layer_body(8, 16, 32), 8, 32, 16, 32,
                     agg2.reshape(8, _N, 32), deg2, b2.reshape(1, 256), W3)
    spmm32x16 = _make_spmm(16, 32)
    agg3 = spmm32x16(y3.reshape(16 * _N, 32), srcq, dst_p)

    batch2 = batch.reshape(_N, 1)
    gs, gmp = pl.pallas_call(
        _pool_body,
        grid=(_N // _BN,),
        in_specs=[
            pl.BlockSpec((16, _BN, 32), lambda i: (0, i, 0)),
            pl.BlockSpec((_BN, 1), lambda i: (i, 0)),
            pl.BlockSpec((1, 512), lambda i: (0, 0)),
            pl.BlockSpec((_BN, 1), lambda i: (i, 0)),
        ],
        out_specs=[
            pl.BlockSpec((_G, 512), lambda i: (0, 0)),
            pl.BlockSpec((_G, 512), lambda i: (0, 0)),
        ],
        out_shape=[
            jax.ShapeDtypeStruct((_G, 512), jnp.float32),
            jax.ShapeDtypeStruct((_G, 512), jnp.float32),
        ],
    )(agg3.reshape(16, _N, 32), deg2, b3.reshape(1, 512), batch2)

    sf = solvent_fingerprint.reshape(_G, 512)
    out = pl.pallas_call(
        _mlp_body,
        out_shape=jax.ShapeDtypeStruct((_G, 1), jnp.float32),
    )(batch2, gs, gmp, sf,
      Wg1, bg1.reshape(1, -1), Wg2, bg2.reshape(1, -1),
      Ws1, bs1.reshape(1, -1), Ws2, bs2.reshape(1, -1),
      Wf1, bf1.reshape(1, -1), Wf2, bf2.reshape(1, -1), Wo, bo.reshape(1, -1))
    return out


# speedup vs baseline: 1.5167x; 1.5167x over previous
import functools

import jax
import jax.numpy as jnp
from jax import lax
from jax.experimental import pallas as pl
from jax.experimental.pallas import tpu as pltpu
from jax.experimental.pallas import tpu_sc as plsc

_N = 10000
_E = 320000
_G = 64
_NSUB = 16
_B = 128
_TS = 160
_TI = 168
_EPS = _E // _NSUB
_NPAD = 10016
_RPS = 624
_RTAIL = _N - _NSUB * _RPS
_BN = 1000


def _make_spmm(nchunk, fc):
    cpc = nchunk // 2
    mesh = plsc.VectorSubcoreMesh(core_axis_name="c", subcore_axis_name="s")

    @functools.partial(
        pl.kernel,
        out_type=jax.ShapeDtypeStruct((nchunk * _N, fc), jnp.float32),
        mesh=mesh,
        scratch_types=[
            pltpu.VMEM((_TI, _B), jnp.int32),
            pltpu.VMEM((_TI, _B), jnp.int32),
            pltpu.VMEM((_B, fc), jnp.float32),
            pltpu.VMEM_SHARED((_NPAD, fc), jnp.float32),
            pltpu.SemaphoreType.DMA,
        ],
        compiler_params=pltpu.CompilerParams(use_tc_tiling_on_sc=False),
    )
    def spmm(y_hbm, srcq_hbm, dst_hbm, out_hbm, src_v, dst_v, rows, acc, gsem):
        c = lax.axis_index("c")
        s = lax.axis_index("s")
        pltpu.sync_copy(dst_hbm.at[s], dst_v)
        for j in range(cpc):
            q = c * cpc + j
            pltpu.sync_copy(srcq_hbm.at[q, s], src_v)
            pltpu.sync_copy(y_hbm.at[pl.ds(q * _N + s * _RPS, _RPS)],
                            acc.at[pl.ds(s * _RPS, _RPS)])

            @pl.when(s == 0)
            def _():
                pltpu.sync_copy(
                    y_hbm.at[pl.ds(q * _N + _NSUB * _RPS, _RTAIL)],
                    acc.at[pl.ds(_NSUB * _RPS, _RTAIL)])

            plsc.subcore_barrier()

            def rbody(t, carry):
                pltpu.async_copy(y_hbm.at[src_v.at[t]], rows, gsem).wait()
                pltpu.sync_copy(rows, acc.at[dst_v.at[t]], add=True)
                return carry

            lax.fori_loop(0, _TS, rbody, 0)

            plsc.subcore_barrier()
            pltpu.sync_copy(acc.at[pl.ds(s * _RPS, _RPS)],
                            out_hbm.at[pl.ds(q * _N + s * _RPS, _RPS)])

            @pl.when(s == 0)
            def _():
                pltpu.sync_copy(
                    acc.at[pl.ds(_NSUB * _RPS, _RTAIL)],
                    out_hbm.at[pl.ds(q * _N + _NSUB * _RPS, _RTAIL)])

            if j + 1 < cpc:
                plsc.subcore_barrier()

    return spmm


def _l1_body(x_ref, deg_ref, w_ref, out_ref):
    dinv = lax.rsqrt(deg_ref[...])
    y = jnp.dot(x_ref[...] * dinv, w_ref[...],
                preferred_element_type=jnp.float32)
    for q in range(2):
        out_ref[q] = y[:, q * 64:(q + 1) * 64]


def _make_layer_body(nc_in, nc_out, fco):
    def body(a_ref, deg_ref, b_ref, w_ref, out_ref):
        dinv = lax.rsqrt(deg_ref[...])
        h = jnp.concatenate([a_ref[i] for i in range(nc_in)], axis=1)
        h = jax.nn.relu(h * dinv + b_ref[...])
        y = jnp.dot(h * dinv, w_ref[...], preferred_element_type=jnp.float32)
        for q in range(nc_out):
            out_ref[q] = y[:, q * fco:(q + 1) * fco]
    return body


def _pool_body(a_ref, deg_ref, b_ref, batch_ref, gs_ref, gmp_ref):
    i = pl.program_id(0)
    dinv = lax.rsqrt(deg_ref[...])
    h = jnp.concatenate([a_ref[q] for q in range(8)], axis=1)
    h = jax.nn.relu(h * dinv + b_ref[...])
    gid = lax.broadcasted_iota(jnp.int32, (1, _G), 1)
    onehot = (batch_ref[...] == gid).astype(jnp.float32)
    gs = lax.dot_general(onehot, h, (((0,), (0,)), ((), ())),
                         preferred_element_type=jnp.float32)
    parts = []
    for g in range(_G):
        parts.append(jnp.max(onehot[:, g:g + 1] * h, axis=0, keepdims=True))
    gmp = jnp.concatenate(parts, axis=0)

    @pl.when(i == 0)
    def _():
        gs_ref[...] = gs
        gmp_ref[...] = gmp

    @pl.when(i > 0)
    def _():
        gs_ref[...] += gs
        gmp_ref[...] = jnp.maximum(gmp_ref[...], gmp)


def _mlp_body(batch_ref, gs_ref, gmp_ref, sf_ref,
              Wg1_ref, bg1_ref, Wg2_ref, bg2_ref,
              Ws1_ref, bs1_ref, Ws2_ref, bs2_ref,
              Wf1_ref, bf1_ref, Wf2_ref, bf2_ref, Wo_ref, bo_ref, out_ref):
    gid = lax.broadcasted_iota(jnp.int32, (1, _G), 1)
    onehot = (batch_ref[...] == gid).astype(jnp.float32)
    ones = jnp.ones((_N, 1), jnp.float32)
    counts = lax.dot_general(onehot, ones, (((0,), (0,)), ((), ())),
                             preferred_element_type=jnp.float32)
    gap = gs_ref[...] / jnp.maximum(counts, 1.0)
    comb = jnp.concatenate([gap, gmp_ref[...]], axis=1)
    comb = jax.nn.relu(
        jnp.dot(comb, Wg1_ref[...], preferred_element_type=jnp.float32)
        + bg1_ref[...])
    comb = jax.nn.relu(
        jnp.dot(comb, Wg2_ref[...], preferred_element_type=jnp.float32)
        + bg2_ref[...])
    s = jax.nn.relu(
        jnp.dot(sf_ref[...], Ws1_ref[...], preferred_element_type=jnp.float32)
        + bs1_ref[...])
    s = jax.nn.relu(
        jnp.dot(s, Ws2_ref[...], preferred_element_type=jnp.float32)
        + bs2_ref[...])
    z = jnp.concatenate([comb, s], axis=1)
    z = jax.nn.relu(
        jnp.dot(z, Wf1_ref[...], preferred_element_type=jnp.float32)
        + bf1_ref[...])
    z = jax.nn.relu(
        jnp.dot(z, Wf2_ref[...], preferred_element_type=jnp.float32)
        + bf2_ref[...])
    out_ref[...] = (
        jnp.dot(z, Wo_ref[...], preferred_element_type=jnp.float32)
        + bo_ref[...])


def _layer_call(body, nc_in, fci, nc_out, fco, a, deg2, b, w):
    return pl.pallas_call(
        body,
        grid=(_N // _BN,),
        in_specs=[
            pl.BlockSpec((nc_in, _BN, fci), lambda i: (0, i, 0)),
            pl.BlockSpec((_BN, 1), lambda i: (i, 0)),
            pl.BlockSpec((1, nc_in * fci), lambda i: (0, 0)),
            pl.BlockSpec((nc_in * fci, nc_out * fco), lambda i: (0, 0)),
        ],
        out_specs=pl.BlockSpec((nc_out, _BN, fco), lambda i: (0, i, 0)),
        out_shape=jax.ShapeDtypeStruct((nc_out, _N, fco), jnp.float32),
    )(a, deg2, b, w)


def kernel(x, edge_index, edge_attr, batch, solvent_fingerprint,
           W1, b1, W2, b2, W3, b3, Wg1, bg1, Wg2, bg2,
           Ws1, bs1, Ws2, bs2, Wf1, bf1, Wf2, bf2, Wo, bo):
    src = edge_index[0]
    dst = edge_index[1]
    padw = _TI * _B - _EPS
    src16 = jnp.pad(src.reshape(_NSUB, _EPS), ((0, 0), (0, padw)))
    qoff = (jnp.arange(2, dtype=jnp.int32) * _N)[:, None, None, None]
    srcq = src16.reshape(1, _NSUB, _TI, _B) + qoff
    dst_p = jnp.pad(dst.reshape(_NSUB, _EPS), ((0, 0), (0, padw)),
                    constant_values=_N).reshape(_NSUB, _TI, _B)

    spmm16 = _make_spmm(2, 16)
    deg_full = spmm16(jnp.ones((2 * _N, 16), jnp.float32), srcq, dst_p)
    deg2 = deg_full[:_N, :1]
    spmm64 = _make_spmm(2, 64)

    y1 = pl.pallas_call(
        _l1_body,
        grid=(_N // _BN,),
        in_specs=[
            pl.BlockSpec((_BN, 128), lambda i: (i, 0)),
            pl.BlockSpec((_BN, 1), lambda i: (i, 0)),
            pl.BlockSpec((128, 128), lambda i: (0, 0)),
        ],
        out_specs=pl.BlockSpec((2, _BN, 64), lambda i: (0, i, 0)),
        out_shape=jax.ShapeDtypeStruct((2, _N, 64), jnp.float32),
    )(x, deg2, W1)
    agg1 = spmm64(y1.reshape(2 * _N, 64), srcq, dst_p)

    y2 = _layer_call(_make_layer_body(2, 4, 64), 2, 64, 4, 64,
                     agg1.reshape(2, _N, 64), deg2, b1.reshape(1, 128), W2)
    y2f = y2.reshape(4 * _N, 64)
    agg2 = jnp.concatenate([spmm64(y2f[:2 * _N], srcq, dst_p),
                            spmm64(y2f[2 * _N:], srcq, dst_p)])

    y3 = _layer_call(_make_layer_body(4, 8, 64), 4, 64, 8, 64,
                     agg2.reshape(4, _N, 64), deg2, b2.reshape(1, 256), W3)
    y3f = y3.reshape(8 * _N, 64)
    agg3 = jnp.concatenate(
        [spmm64(y3f[i * 2 * _N:(i + 1) * 2 * _N], srcq, dst_p)
         for i in range(4)])

    batch2 = batch.reshape(_N, 1)
    gs, gmp = pl.pallas_call(
        _pool_body,
        grid=(_N // _BN,),
        in_specs=[
            pl.BlockSpec((8, _BN, 64), lambda i: (0, i, 0)),
            pl.BlockSpec((_BN, 1), lambda i: (i, 0)),
            pl.BlockSpec((1, 512), lambda i: (0, 0)),
            pl.BlockSpec((_BN, 1), lambda i: (i, 0)),
        ],
        out_specs=[
            pl.BlockSpec((_G, 512), lambda i: (0, 0)),
            pl.BlockSpec((_G, 512), lambda i: (0, 0)),
        ],
        out_shape=[
            jax.ShapeDtypeStruct((_G, 512), jnp.float32),
            jax.ShapeDtypeStruct((_G, 512), jnp.float32),
        ],
    )(agg3.reshape(8, _N, 64), deg2, b3.reshape(1, 512), batch2)

    sf = solvent_fingerprint.reshape(_G, 512)
    out = pl.pallas_call(
        _mlp_body,
        out_shape=jax.ShapeDtypeStruct((_G, 1), jnp.float32),
    )(batch2, gs, gmp, sf,
      Wg1, bg1.reshape(1, -1), Wg2, bg2.reshape(1, -1),
      Ws1, bs1.reshape(1, -1), Ws2, bs2.reshape(1, -1),
      Wf1, bf1.reshape(1, -1), Wf2, bf2.reshape(1, -1), Wo, bo.reshape(1, -1))
    return out


# speedup vs baseline: 1.5340x; 1.0114x over previous
import functools

import jax
import jax.numpy as jnp
from jax import lax
from jax.experimental import pallas as pl
from jax.experimental.pallas import tpu as pltpu
from jax.experimental.pallas import tpu_sc as plsc

_N = 10000
_E = 320000
_G = 64
_NSUB = 16
_B = 128
_TS = 160
_TI = 168
_EPS = _E // _NSUB
_NPAD = 10016
_RPS = 624
_RTAIL = _N - _NSUB * _RPS
_BN = 1000


def _make_spmm(nchunk, fc):
    cpc = nchunk // 2
    mesh = plsc.VectorSubcoreMesh(core_axis_name="c", subcore_axis_name="s")

    @functools.partial(
        pl.kernel,
        out_type=jax.ShapeDtypeStruct((nchunk * _N, fc), jnp.float32),
        mesh=mesh,
        scratch_types=[
            pltpu.VMEM((_TI, _B), jnp.int32),
            pltpu.VMEM((_TI, _B), jnp.int32),
            pltpu.VMEM((_B, fc), jnp.float32),
            pltpu.VMEM_SHARED((_NPAD, fc), jnp.float32),
            pltpu.SemaphoreType.DMA,
        ],
        compiler_params=pltpu.CompilerParams(use_tc_tiling_on_sc=False),
    )
    def spmm(y_hbm, srcq_hbm, dst_hbm, out_hbm, src_v, dst_v, rows, acc, gsem):
        c = lax.axis_index("c")
        s = lax.axis_index("s")
        pltpu.sync_copy(dst_hbm.at[s], dst_v)
        for j in range(cpc):
            q = c * cpc + j
            pltpu.sync_copy(srcq_hbm.at[q, s], src_v)
            pltpu.sync_copy(y_hbm.at[pl.ds(q * _N + s * _RPS, _RPS)],
                            acc.at[pl.ds(s * _RPS, _RPS)])

            @pl.when(s == 0)
            def _():
                pltpu.sync_copy(
                    y_hbm.at[pl.ds(q * _N + _NSUB * _RPS, _RTAIL)],
                    acc.at[pl.ds(_NSUB * _RPS, _RTAIL)])

            plsc.subcore_barrier()

            def rbody(t, carry):
                pltpu.async_copy(y_hbm.at[src_v.at[t]], rows, gsem).wait()
                pltpu.sync_copy(rows, acc.at[dst_v.at[t]], add=True)
                return carry

            lax.fori_loop(0, _TS, rbody, 0)

            plsc.subcore_barrier()
            pltpu.sync_copy(acc.at[pl.ds(s * _RPS, _RPS)],
                            out_hbm.at[pl.ds(q * _N + s * _RPS, _RPS)])

            @pl.when(s == 0)
            def _():
                pltpu.sync_copy(
                    acc.at[pl.ds(_NSUB * _RPS, _RTAIL)],
                    out_hbm.at[pl.ds(q * _N + _NSUB * _RPS, _RTAIL)])

            if j + 1 < cpc:
                plsc.subcore_barrier()

    return spmm


def _l1_body(x_ref, deg_ref, w_ref, out_ref):
    dinv = lax.rsqrt(deg_ref[...])
    y = jnp.dot(x_ref[...] * dinv, w_ref[...],
                preferred_element_type=jnp.float32)
    for q in range(2):
        out_ref[q] = y[:, q * 64:(q + 1) * 64]


def _make_layer_body(nc_in, nc_out, fco):
    def body(a_ref, deg_ref, b_ref, w_ref, out_ref):
        dinv = lax.rsqrt(deg_ref[...])
        h = jnp.concatenate([a_ref[i] for i in range(nc_in)], axis=1)
        h = jax.nn.relu(h * dinv + b_ref[...])
        y = jnp.dot(h * dinv, w_ref[...], preferred_element_type=jnp.float32)
        for q in range(nc_out):
            out_ref[q] = y[:, q * fco:(q + 1) * fco]
    return body


def _pool_body(a_ref, deg_ref, b_ref, batch_ref, gs_ref, gmp_ref):
    i = pl.program_id(0)
    dinv = lax.rsqrt(deg_ref[...])
    h = jnp.concatenate([a_ref[q] for q in range(8)], axis=1)
    h = jax.nn.relu(h * dinv + b_ref[...])
    gid = lax.broadcasted_iota(jnp.int32, (1, _G), 1)
    onehot = (batch_ref[...] == gid).astype(jnp.float32)
    gs = lax.dot_general(onehot, h, (((0,), (0,)), ((), ())),
                         preferred_element_type=jnp.float32)
    parts = []
    for g in range(_G):
        parts.append(jnp.max(onehot[:, g:g + 1] * h, axis=0, keepdims=True))
    gmp = jnp.concatenate(parts, axis=0)

    @pl.when(i == 0)
    def _():
        gs_ref[...] = gs
        gmp_ref[...] = gmp

    @pl.when(i > 0)
    def _():
        gs_ref[...] += gs
        gmp_ref[...] = jnp.maximum(gmp_ref[...], gmp)


def _mlp_body(batch_ref, gs_ref, gmp_ref, sf_ref,
              Wg1_ref, bg1_ref, Wg2_ref, bg2_ref,
              Ws1_ref, bs1_ref, Ws2_ref, bs2_ref,
              Wf1_ref, bf1_ref, Wf2_ref, bf2_ref, Wo_ref, bo_ref, out_ref):
    gid = lax.broadcasted_iota(jnp.int32, (1, _G), 1)
    onehot = (batch_ref[...] == gid).astype(jnp.float32)
    ones = jnp.ones((_N, 1), jnp.float32)
    counts = lax.dot_general(onehot, ones, (((0,), (0,)), ((), ())),
                             preferred_element_type=jnp.float32)
    gap = gs_ref[...] / jnp.maximum(counts, 1.0)
    comb = jnp.concatenate([gap, gmp_ref[...]], axis=1)
    comb = jax.nn.relu(
        jnp.dot(comb, Wg1_ref[...], preferred_element_type=jnp.float32)
        + bg1_ref[...])
    comb = jax.nn.relu(
        jnp.dot(comb, Wg2_ref[...], preferred_element_type=jnp.float32)
        + bg2_ref[...])
    s = jax.nn.relu(
        jnp.dot(sf_ref[...], Ws1_ref[...], preferred_element_type=jnp.float32)
        + bs1_ref[...])
    s = jax.nn.relu(
        jnp.dot(s, Ws2_ref[...], preferred_element_type=jnp.float32)
        + bs2_ref[...])
    z = jnp.concatenate([comb, s], axis=1)
    z = jax.nn.relu(
        jnp.dot(z, Wf1_ref[...], preferred_element_type=jnp.float32)
        + bf1_ref[...])
    z = jax.nn.relu(
        jnp.dot(z, Wf2_ref[...], preferred_element_type=jnp.float32)
        + bf2_ref[...])
    out_ref[...] = (
        jnp.dot(z, Wo_ref[...], preferred_element_type=jnp.float32)
        + bo_ref[...])


def _layer_call(body, nc_in, fci, nc_out, fco, a, deg2, b, w):
    return pl.pallas_call(
        body,
        grid=(_N // _BN,),
        in_specs=[
            pl.BlockSpec((nc_in, _BN, fci), lambda i: (0, i, 0)),
            pl.BlockSpec((_BN, 1), lambda i: (i, 0)),
            pl.BlockSpec((1, nc_in * fci), lambda i: (0, 0)),
            pl.BlockSpec((nc_in * fci, nc_out * fco), lambda i: (0, 0)),
        ],
        out_specs=pl.BlockSpec((nc_out, _BN, fco), lambda i: (0, i, 0)),
        out_shape=jax.ShapeDtypeStruct((nc_out, _N, fco), jnp.float32),
    )(a, deg2, b, w)


def kernel(x, edge_index, edge_attr, batch, solvent_fingerprint,
           W1, b1, W2, b2, W3, b3, Wg1, bg1, Wg2, bg2,
           Ws1, bs1, Ws2, bs2, Wf1, bf1, Wf2, bf2, Wo, bo):
    src = edge_index[0]
    dst = edge_index[1]
    padw = _TI * _B - _EPS
    src16 = jnp.pad(src.reshape(_NSUB, _EPS), ((0, 0), (0, padw)))
    qoff = (jnp.arange(8, dtype=jnp.int32) * _N)[:, None, None, None]
    srcq = src16.reshape(1, _NSUB, _TI, _B) + qoff
    dst_p = jnp.pad(dst.reshape(_NSUB, _EPS), ((0, 0), (0, padw)),
                    constant_values=_N).reshape(_NSUB, _TI, _B)

    spmm16 = _make_spmm(2, 16)
    deg_full = spmm16(jnp.ones((2 * _N, 16), jnp.float32), srcq, dst_p)
    deg2 = deg_full[:_N, :1]
    spmm64a = _make_spmm(2, 64)
    spmm64b = _make_spmm(4, 64)
    spmm64c = _make_spmm(8, 64)

    y1 = pl.pallas_call(
        _l1_body,
        grid=(_N // _BN,),
        in_specs=[
            pl.BlockSpec((_BN, 128), lambda i: (i, 0)),
            pl.BlockSpec((_BN, 1), lambda i: (i, 0)),
            pl.BlockSpec((128, 128), lambda i: (0, 0)),
        ],
        out_specs=pl.BlockSpec((2, _BN, 64), lambda i: (0, i, 0)),
        out_shape=jax.ShapeDtypeStruct((2, _N, 64), jnp.float32),
    )(x, deg2, W1)
    agg1 = spmm64a(y1.reshape(2 * _N, 64), srcq, dst_p)

    y2 = _layer_call(_make_layer_body(2, 4, 64), 2, 64, 4, 64,
                     agg1.reshape(2, _N, 64), deg2, b1.reshape(1, 128), W2)
    agg2 = spmm64b(y2.reshape(4 * _N, 64), srcq, dst_p)

    y3 = _layer_call(_make_layer_body(4, 8, 64), 4, 64, 8, 64,
                     agg2.reshape(4, _N, 64), deg2, b2.reshape(1, 256), W3)
    agg3 = spmm64c(y3.reshape(8 * _N, 64), srcq, dst_p)

    batch2 = batch.reshape(_N, 1)
    gs, gmp = pl.pallas_call(
        _pool_body,
        grid=(_N // _BN,),
        in_specs=[
            pl.BlockSpec((8, _BN, 64), lambda i: (0, i, 0)),
            pl.BlockSpec((_BN, 1), lambda i: (i, 0)),
            pl.BlockSpec((1, 512), lambda i: (0, 0)),
            pl.BlockSpec((_BN, 1), lambda i: (i, 0)),
        ],
        out_specs=[
            pl.BlockSpec((_G, 512), lambda i: (0, 0)),
            pl.BlockSpec((_G, 512), lambda i: (0, 0)),
        ],
        out_shape=[
            jax.ShapeDtypeStruct((_G, 512), jnp.float32),
            jax.ShapeDtypeStruct((_G, 512), jnp.float32),
        ],
    )(agg3.reshape(8, _N, 64), deg2, b3.reshape(1, 512), batch2)

    sf = solvent_fingerprint.reshape(_G, 512)
    out = pl.pallas_call(
        _mlp_body,
        out_shape=jax.ShapeDtypeStruct((_G, 1), jnp.float32),
    )(batch2, gs, gmp, sf,
      Wg1, bg1.reshape(1, -1), Wg2, bg2.reshape(1, -1),
      Ws1, bs1.reshape(1, -1), Ws2, bs2.reshape(1, -1),
      Wf1, bf1.reshape(1, -1), Wf2, bf2.reshape(1, -1), Wo, bo.reshape(1, -1))
    return out


# speedup vs baseline: 1.9571x; 1.2758x over previous
import functools

import jax
import jax.numpy as jnp
from jax import lax
from jax.experimental import pallas as pl
from jax.experimental.pallas import tpu as pltpu
from jax.experimental.pallas import tpu_sc as plsc

_N = 10000
_E = 320000
_G = 64
_NSUB = 16
_B = 128
_TB = 157
_E2 = _NSUB * _TB * _B
_NPAD = 10016
_RPS = 624
_RTAIL = _N - _NSUB * _RPS
_BN = 1000


def _make_spmm(nchunk, fc):
    cpc = nchunk // 2
    mesh = plsc.VectorSubcoreMesh(core_axis_name="c", subcore_axis_name="s")

    @functools.partial(
        pl.kernel,
        out_type=jax.ShapeDtypeStruct((nchunk * _N, fc), jnp.float32),
        mesh=mesh,
        scratch_types=[
            pltpu.VMEM((_TB, _B), jnp.int32),
            pltpu.VMEM((_TB, _B), jnp.int32),
            pltpu.VMEM((_B, fc), jnp.float32),
            pltpu.VMEM_SHARED((_NPAD, fc), jnp.float32),
            pltpu.SemaphoreType.DMA,
        ],
        compiler_params=pltpu.CompilerParams(use_tc_tiling_on_sc=False),
    )
    def spmm(y_hbm, srcq_hbm, dst_hbm, out_hbm, src_v, dst_v, rows, acc, gsem):
        c = lax.axis_index("c")
        s = lax.axis_index("s")
        pltpu.sync_copy(dst_hbm.at[s], dst_v)
        for j in range(cpc):
            q = c * cpc + j
            pltpu.sync_copy(srcq_hbm.at[q, s], src_v)
            pltpu.sync_copy(y_hbm.at[pl.ds(q * _N + s * _RPS, _RPS)],
                            acc.at[pl.ds(s * _RPS, _RPS)])

            @pl.when(s == 0)
            def _():
                pltpu.sync_copy(
                    y_hbm.at[pl.ds(q * _N + _NSUB * _RPS, _RTAIL)],
                    acc.at[pl.ds(_NSUB * _RPS, _RTAIL)])

            plsc.subcore_barrier()

            def rbody(t, carry):
                pltpu.async_copy(y_hbm.at[src_v.at[t]], rows, gsem).wait()
                pltpu.sync_copy(rows, acc.at[dst_v.at[t]], add=True)
                return carry

            lax.fori_loop(0, _TB, rbody, 0)

            plsc.subcore_barrier()
            pltpu.sync_copy(acc.at[pl.ds(s * _RPS, _RPS)],
                            out_hbm.at[pl.ds(q * _N + s * _RPS, _RPS)])

            @pl.when(s == 0)
            def _():
                pltpu.sync_copy(
                    acc.at[pl.ds(_NSUB * _RPS, _RTAIL)],
                    out_hbm.at[pl.ds(q * _N + _NSUB * _RPS, _RTAIL)])

            if j + 1 < cpc:
                plsc.subcore_barrier()

    return spmm


def _l1_body(x_ref, deg_ref, w_ref, out_ref):
    dinv = lax.rsqrt(deg_ref[...])
    y = jnp.dot(x_ref[...] * dinv, w_ref[...],
                preferred_element_type=jnp.float32)
    for q in range(4):
        out_ref[q] = y[:, q * 32:(q + 1) * 32]


def _make_layer_body(nc_in, nc_out, fco):
    def body(a_ref, deg_ref, b_ref, w_ref, out_ref):
        dinv = lax.rsqrt(deg_ref[...])
        h = jnp.concatenate([a_ref[i] for i in range(nc_in)], axis=1)
        h = jax.nn.relu(h * dinv + b_ref[...])
        y = jnp.dot(h * dinv, w_ref[...], preferred_element_type=jnp.float32)
        for q in range(nc_out):
            out_ref[q] = y[:, q * fco:(q + 1) * fco]
    return body


def _pool_body(a_ref, deg_ref, b_ref, batch_ref, gs_ref, gmp_ref):
    i = pl.program_id(0)
    dinv = lax.rsqrt(deg_ref[...])
    h = jnp.concatenate([a_ref[q] for q in range(8)], axis=1)
    h = jax.nn.relu(h * dinv + b_ref[...])
    gid = lax.broadcasted_iota(jnp.int32, (1, _G), 1)
    onehot = (batch_ref[...] == gid).astype(jnp.float32)
    gs = lax.dot_general(onehot, h, (((0,), (0,)), ((), ())),
                         preferred_element_type=jnp.float32)
    parts = []
    for g in range(_G):
        parts.append(jnp.max(onehot[:, g:g + 1] * h, axis=0, keepdims=True))
    gmp = jnp.concatenate(parts, axis=0)

    @pl.when(i == 0)
    def _():
        gs_ref[...] = gs
        gmp_ref[...] = gmp

    @pl.when(i > 0)
    def _():
        gs_ref[...] += gs
        gmp_ref[...] = jnp.maximum(gmp_ref[...], gmp)


def _mlp_body(batch_ref, gs_ref, gmp_ref, sf_ref,
              Wg1_ref, bg1_ref, Wg2_ref, bg2_ref,
              Ws1_ref, bs1_ref, Ws2_ref, bs2_ref,
              Wf1_ref, bf1_ref, Wf2_ref, bf2_ref, Wo_ref, bo_ref, out_ref):
    gid = lax.broadcasted_iota(jnp.int32, (1, _G), 1)
    onehot = (batch_ref[...] == gid).astype(jnp.float32)
    ones = jnp.ones((_N, 1), jnp.float32)
    counts = lax.dot_general(onehot, ones, (((0,), (0,)), ((), ())),
                             preferred_element_type=jnp.float32)
    gap = gs_ref[...] / jnp.maximum(counts, 1.0)
    comb = jnp.concatenate([gap, gmp_ref[...]], axis=1)
    comb = jax.nn.relu(
        jnp.dot(comb, Wg1_ref[...], preferred_element_type=jnp.float32)
        + bg1_ref[...])
    comb = jax.nn.relu(
        jnp.dot(comb, Wg2_ref[...], preferred_element_type=jnp.float32)
        + bg2_ref[...])
    s = jax.nn.relu(
        jnp.dot(sf_ref[...], Ws1_ref[...], preferred_element_type=jnp.float32)
        + bs1_ref[...])
    s = jax.nn.relu(
        jnp.dot(s, Ws2_ref[...], preferred_element_type=jnp.float32)
        + bs2_ref[...])
    z = jnp.concatenate([comb, s], axis=1)
    z = jax.nn.relu(
        jnp.dot(z, Wf1_ref[...], preferred_element_type=jnp.float32)
        + bf1_ref[...])
    z = jax.nn.relu(
        jnp.dot(z, Wf2_ref[...], preferred_element_type=jnp.float32)
        + bf2_ref[...])
    out_ref[...] = (
        jnp.dot(z, Wo_ref[...], preferred_element_type=jnp.float32)
        + bo_ref[...])


def _layer_call(body, nc_in, fci, nc_out, fco, a, deg2, b, w):
    return pl.pallas_call(
        body,
        grid=(_N // _BN,),
        in_specs=[
            pl.BlockSpec((nc_in, _BN, fci), lambda i: (0, i, 0)),
            pl.BlockSpec((_BN, 1), lambda i: (i, 0)),
            pl.BlockSpec((1, nc_in * fci), lambda i: (0, 0)),
            pl.BlockSpec((nc_in * fci, nc_out * fco), lambda i: (0, 0)),
        ],
        out_specs=pl.BlockSpec((nc_out, _BN, fco), lambda i: (0, i, 0)),
        out_shape=jax.ShapeDtypeStruct((nc_out, _N, fco), jnp.float32),
    )(a, deg2, b, w)


def kernel(x, edge_index, edge_attr, batch, solvent_fingerprint,
           W1, b1, W2, b2, W3, b3, Wg1, bg1, Wg2, bg2,
           Ws1, bs1, Ws2, bs2, Wf1, bf1, Wf2, bf2, Wo, bo):
    src = edge_index[0]
    dst = edge_index[1]
    src_p = jnp.concatenate([src, jnp.zeros((_E2 - _E,), jnp.int32)])
    qoff = (jnp.arange(8, dtype=jnp.int32) * _N)[:, None, None, None]
    srcq = src_p.reshape(1, _NSUB, _TB, _B) + qoff
    dst_p = jnp.concatenate(
        [dst, jnp.full((_E2 - _E,), _N, jnp.int32)]).reshape(_NSUB, _TB, _B)

    spmm16 = _make_spmm(2, 16)
    deg_full = spmm16(jnp.ones((2 * _N, 16), jnp.float32), srcq, dst_p)
    deg2 = deg_full[:_N, :1]
    spmm32 = _make_spmm(4, 32)
    spmm64 = _make_spmm(4, 64)
    spmm64x8 = _make_spmm(8, 64)

    y1 = pl.pallas_call(
        _l1_body,
        grid=(_N // _BN,),
        in_specs=[
            pl.BlockSpec((_BN, 128), lambda i: (i, 0)),
            pl.BlockSpec((_BN, 1), lambda i: (i, 0)),
            pl.BlockSpec((128, 128), lambda i: (0, 0)),
        ],
        out_specs=pl.BlockSpec((4, _BN, 32), lambda i: (0, i, 0)),
        out_shape=jax.ShapeDtypeStruct((4, _N, 32), jnp.float32),
    )(x, deg2, W1)
    agg1 = spmm32(y1.reshape(4 * _N, 32), srcq, dst_p)

    y2 = _layer_call(_make_layer_body(4, 4, 64), 4, 32, 4, 64,
                     agg1.reshape(4, _N, 32), deg2, b1.reshape(1, 128), W2)
    agg2 = spmm64(y2.reshape(4 * _N, 64), srcq, dst_p)

    y3 = _layer_call(_make_layer_body(4, 8, 64), 4, 64, 8, 64,
                     agg2.reshape(4, _N, 64), deg2, b2.reshape(1, 256), W3)
    agg3 = spmm64x8(y3.reshape(8 * _N, 64), srcq, dst_p)

    batch2 = batch.reshape(_N, 1)
    gs, gmp = pl.pallas_call(
        _pool_body,
        grid=(_N // _BN,),
        in_specs=[
            pl.BlockSpec((8, _BN, 64), lambda i: (0, i, 0)),
            pl.BlockSpec((_BN, 1), lambda i: (i, 0)),
            pl.BlockSpec((1, 512), lambda i: (0, 0)),
            pl.BlockSpec((_BN, 1), lambda i: (i, 0)),
        ],
        out_specs=[
            pl.BlockSpec((_G, 512), lambda i: (0, 0)),
            pl.BlockSpec((_G, 512), lambda i: (0, 0)),
        ],
        out_shape=[
            jax.ShapeDtypeStruct((_G, 512), jnp.float32),
            jax.ShapeDtypeStruct((_G, 512), jnp.float32),
        ],
    )(agg3.reshape(8, _N, 64), deg2, b3.reshape(1, 512), batch2)

    sf = solvent_fingerprint.reshape(_G, 512)
    out = pl.pallas_call(
        _mlp_body,
        out_shape=jax.ShapeDtypeStruct((_G, 1), jnp.float32),
    )(batch2, gs, gmp, sf,
      Wg1, bg1.reshape(1, -1), Wg2, bg2.reshape(1, -1),
      Ws1, bs1.reshape(1, -1), Ws2, bs2.reshape(1, -1),
      Wf1, bf1.reshape(1, -1), Wf2, bf2.reshape(1, -1), Wo, bo.reshape(1, -1))
    return out


# speedup vs baseline: 2.3572x; 1.2044x over previous
import functools

import jax
import jax.numpy as jnp
from jax import lax
from jax.experimental import pallas as pl
from jax.experimental.pallas import tpu as pltpu
from jax.experimental.pallas import tpu_sc as plsc

_N = 10000
_E = 320000
_G = 64
_NSUB = 16
_B = 128
_TB = 157
_E2 = _NSUB * _TB * _B
_NPAD = 10016
_RPS = 624
_RTAIL = _N - _NSUB * _RPS
_BN = 1000


def _make_spmm(nchunk, fc):
    cpc = nchunk // 2
    mesh = plsc.VectorSubcoreMesh(core_axis_name="c", subcore_axis_name="s")

    @functools.partial(
        pl.kernel,
        out_type=jax.ShapeDtypeStruct((nchunk * _N, fc), jnp.float32),
        mesh=mesh,
        scratch_types=[
            pltpu.VMEM((_TB, _B), jnp.int32),
            pltpu.VMEM((_TB, _B), jnp.int32),
            [pltpu.VMEM((_B, fc), jnp.float32) for _ in range(2)],
            pltpu.VMEM_SHARED((_NPAD, fc), jnp.float32),
            pltpu.SemaphoreType.DMA,
            pltpu.SemaphoreType.DMA,
        ],
        compiler_params=pltpu.CompilerParams(use_tc_tiling_on_sc=False),
    )
    def spmm(y_hbm, srcq_hbm, dst_hbm, out_hbm, src_v, dst_v, rows, acc,
             gsem0, gsem1):
        c = lax.axis_index("c")
        s = lax.axis_index("s")
        gsem = (gsem0, gsem1)

        def g_issue(b, t):
            pltpu.async_copy(y_hbm.at[src_v.at[t]], rows[b], gsem[b])

        def g_wait(b, t):
            pltpu.make_async_copy(y_hbm.at[src_v.at[t]], rows[b],
                                  gsem[b]).wait()
        pltpu.sync_copy(dst_hbm.at[s], dst_v)
        for j in range(cpc):
            q = c * cpc + j
            pltpu.sync_copy(srcq_hbm.at[q, s], src_v)
            pltpu.sync_copy(y_hbm.at[pl.ds(q * _N + s * _RPS, _RPS)],
                            acc.at[pl.ds(s * _RPS, _RPS)])

            @pl.when(s == 0)
            def _():
                pltpu.sync_copy(
                    y_hbm.at[pl.ds(q * _N + _NSUB * _RPS, _RTAIL)],
                    acc.at[pl.ds(_NSUB * _RPS, _RTAIL)])

            plsc.subcore_barrier()

            g_issue(0, 0)

            def rbody(rr, carry):
                r0 = rr * 2
                g_wait(0, r0)
                g_issue(1, r0 + 1)
                pltpu.sync_copy(rows[0], acc.at[dst_v.at[r0]], add=True)
                g_wait(1, r0 + 1)
                g_issue(0, r0 + 2)
                pltpu.sync_copy(rows[1], acc.at[dst_v.at[r0 + 1]], add=True)
                return carry

            lax.fori_loop(0, _TB // 2, rbody, 0)
            g_wait(0, _TB - 1)
            pltpu.sync_copy(rows[0], acc.at[dst_v.at[_TB - 1]], add=True)

            plsc.subcore_barrier()
            pltpu.sync_copy(acc.at[pl.ds(s * _RPS, _RPS)],
                            out_hbm.at[pl.ds(q * _N + s * _RPS, _RPS)])

            @pl.when(s == 0)
            def _():
                pltpu.sync_copy(
                    acc.at[pl.ds(_NSUB * _RPS, _RTAIL)],
                    out_hbm.at[pl.ds(q * _N + _NSUB * _RPS, _RTAIL)])

            if j + 1 < cpc:
                plsc.subcore_barrier()

    return spmm


def _l1_body(x_ref, deg_ref, w_ref, out_ref):
    dinv = lax.rsqrt(deg_ref[...])
    y = jnp.dot(x_ref[...] * dinv, w_ref[...],
                preferred_element_type=jnp.float32)
    for q in range(4):
        out_ref[q] = y[:, q * 32:(q + 1) * 32]


def _make_layer_body(nc_in, nc_out, fco):
    def body(a_ref, deg_ref, b_ref, w_ref, out_ref):
        dinv = lax.rsqrt(deg_ref[...])
        h = jnp.concatenate([a_ref[i] for i in range(nc_in)], axis=1)
        h = jax.nn.relu(h * dinv + b_ref[...])
        y = jnp.dot(h * dinv, w_ref[...], preferred_element_type=jnp.float32)
        for q in range(nc_out):
            out_ref[q] = y[:, q * fco:(q + 1) * fco]
    return body


def _pool_body(a_ref, deg_ref, b_ref, batch_ref, gs_ref, gmp_ref):
    i = pl.program_id(0)
    dinv = lax.rsqrt(deg_ref[...])
    h = jnp.concatenate([a_ref[q] for q in range(8)], axis=1)
    h = jax.nn.relu(h * dinv + b_ref[...])
    gid = lax.broadcasted_iota(jnp.int32, (1, _G), 1)
    onehot = (batch_ref[...] == gid).astype(jnp.float32)
    gs = lax.dot_general(onehot, h, (((0,), (0,)), ((), ())),
                         preferred_element_type=jnp.float32)
    parts = []
    for g in range(_G):
        parts.append(jnp.max(onehot[:, g:g + 1] * h, axis=0, keepdims=True))
    gmp = jnp.concatenate(parts, axis=0)

    @pl.when(i == 0)
    def _():
        gs_ref[...] = gs
        gmp_ref[...] = gmp

    @pl.when(i > 0)
    def _():
        gs_ref[...] += gs
        gmp_ref[...] = jnp.maximum(gmp_ref[...], gmp)


def _mlp_body(batch_ref, gs_ref, gmp_ref, sf_ref,
              Wg1_ref, bg1_ref, Wg2_ref, bg2_ref,
              Ws1_ref, bs1_ref, Ws2_ref, bs2_ref,
              Wf1_ref, bf1_ref, Wf2_ref, bf2_ref, Wo_ref, bo_ref, out_ref):
    gid = lax.broadcasted_iota(jnp.int32, (1, _G), 1)
    onehot = (batch_ref[...] == gid).astype(jnp.float32)
    ones = jnp.ones((_N, 1), jnp.float32)
    counts = lax.dot_general(onehot, ones, (((0,), (0,)), ((), ())),
                             preferred_element_type=jnp.float32)
    gap = gs_ref[...] / jnp.maximum(counts, 1.0)
    comb = jnp.concatenate([gap, gmp_ref[...]], axis=1)
    comb = jax.nn.relu(
        jnp.dot(comb, Wg1_ref[...], preferred_element_type=jnp.float32)
        + bg1_ref[...])
    comb = jax.nn.relu(
        jnp.dot(comb, Wg2_ref[...], preferred_element_type=jnp.float32)
        + bg2_ref[...])
    s = jax.nn.relu(
        jnp.dot(sf_ref[...], Ws1_ref[...], preferred_element_type=jnp.float32)
        + bs1_ref[...])
    s = jax.nn.relu(
        jnp.dot(s, Ws2_ref[...], preferred_element_type=jnp.float32)
        + bs2_ref[...])
    z = jnp.concatenate([comb, s], axis=1)
    z = jax.nn.relu(
        jnp.dot(z, Wf1_ref[...], preferred_element_type=jnp.float32)
        + bf1_ref[...])
    z = jax.nn.relu(
        jnp.dot(z, Wf2_ref[...], preferred_element_type=jnp.float32)
        + bf2_ref[...])
    out_ref[...] = (
        jnp.dot(z, Wo_ref[...], preferred_element_type=jnp.float32)
        + bo_ref[...])


def _layer_call(body, nc_in, fci, nc_out, fco, a, deg2, b, w):
    return pl.pallas_call(
        body,
        grid=(_N // _BN,),
        in_specs=[
            pl.BlockSpec((nc_in, _BN, fci), lambda i: (0, i, 0)),
            pl.BlockSpec((_BN, 1), lambda i: (i, 0)),
            pl.BlockSpec((1, nc_in * fci), lambda i: (0, 0)),
            pl.BlockSpec((nc_in * fci, nc_out * fco), lambda i: (0, 0)),
        ],
        out_specs=pl.BlockSpec((nc_out, _BN, fco), lambda i: (0, i, 0)),
        out_shape=jax.ShapeDtypeStruct((nc_out, _N, fco), jnp.float32),
    )(a, deg2, b, w)


def kernel(x, edge_index, edge_attr, batch, solvent_fingerprint,
           W1, b1, W2, b2, W3, b3, Wg1, bg1, Wg2, bg2,
           Ws1, bs1, Ws2, bs2, Wf1, bf1, Wf2, bf2, Wo, bo):
    src = edge_index[0]
    dst = edge_index[1]
    src_p = jnp.concatenate([src, jnp.zeros((_E2 - _E,), jnp.int32)])
    qoff = (jnp.arange(8, dtype=jnp.int32) * _N)[:, None, None, None]
    srcq = src_p.reshape(1, _NSUB, _TB, _B) + qoff
    dst_p = jnp.concatenate(
        [dst, jnp.full((_E2 - _E,), _N, jnp.int32)]).reshape(_NSUB, _TB, _B)

    spmm16 = _make_spmm(2, 16)
    deg_full = spmm16(jnp.ones((2 * _N, 16), jnp.float32), srcq, dst_p)
    deg2 = deg_full[:_N, :1]
    spmm32 = _make_spmm(4, 32)
    spmm64 = _make_spmm(4, 64)
    spmm64x8 = _make_spmm(8, 64)

    y1 = pl.pallas_call(
        _l1_body,
        grid=(_N // _BN,),
        in_specs=[
            pl.BlockSpec((_BN, 128), lambda i: (i, 0)),
            pl.BlockSpec((_BN, 1), lambda i: (i, 0)),
            pl.BlockSpec((128, 128), lambda i: (0, 0)),
        ],
        out_specs=pl.BlockSpec((4, _BN, 32), lambda i: (0, i, 0)),
        out_shape=jax.ShapeDtypeStruct((4, _N, 32), jnp.float32),
    )(x, deg2, W1)
    agg1 = spmm32(y1.reshape(4 * _N, 32), srcq, dst_p)

    y2 = _layer_call(_make_layer_body(4, 4, 64), 4, 32, 4, 64,
                     agg1.reshape(4, _N, 32), deg2, b1.reshape(1, 128), W2)
    agg2 = spmm64(y2.reshape(4 * _N, 64), srcq, dst_p)

    y3 = _layer_call(_make_layer_body(4, 8, 64), 4, 64, 8, 64,
                     agg2.reshape(4, _N, 64), deg2, b2.reshape(1, 256), W3)
    agg3 = spmm64x8(y3.reshape(8 * _N, 64), srcq, dst_p)

    batch2 = batch.reshape(_N, 1)
    gs, gmp = pl.pallas_call(
        _pool_body,
        grid=(_N // _BN,),
        in_specs=[
            pl.BlockSpec((8, _BN, 64), lambda i: (0, i, 0)),
            pl.BlockSpec((_BN, 1), lambda i: (i, 0)),
            pl.BlockSpec((1, 512), lambda i: (0, 0)),
            pl.BlockSpec((_BN, 1), lambda i: (i, 0)),
        ],
        out_specs=[
            pl.BlockSpec((_G, 512), lambda i: (0, 0)),
            pl.BlockSpec((_G, 512), lambda i: (0, 0)),
        ],
        out_shape=[
            jax.ShapeDtypeStruct((_G, 512), jnp.float32),
            jax.ShapeDtypeStruct((_G, 512), jnp.float32),
        ],
    )(agg3.reshape(8, _N, 64), deg2, b3.reshape(1, 512), batch2)

    sf = solvent_fingerprint.reshape(_G, 512)
    out = pl.pallas_call(
        _mlp_body,
        out_shape=jax.ShapeDtypeStruct((_G, 1), jnp.float32),
    )(batch2, gs, gmp, sf,
      Wg1, bg1.reshape(1, -1), Wg2, bg2.reshape(1, -1),
      Ws1, bs1.reshape(1, -1), Ws2, bs2.reshape(1, -1),
      Wf1, bf1.reshape(1, -1), Wf2, bf2.reshape(1, -1), Wo, bo.reshape(1, -1))
    return out


# speedup vs baseline: 2.4582x; 1.0428x over previous
import functools

import jax
import jax.numpy as jnp
from jax import lax
from jax.experimental import pallas as pl
from jax.experimental.pallas import tpu as pltpu
from jax.experimental.pallas import tpu_sc as plsc

_N = 10000
_E = 320000
_G = 64
_NSUB = 16
_B = 128
_TB = 157
_E2 = _NSUB * _TB * _B
_NPAD = 10016
_RPS = 624
_RTAIL = _N - _NSUB * _RPS
_BN = 1000


def _make_spmm(nchunk, fc):
    cpc = nchunk // 2
    mesh = plsc.VectorSubcoreMesh(core_axis_name="c", subcore_axis_name="s")

    @functools.partial(
        pl.kernel,
        out_type=jax.ShapeDtypeStruct((nchunk * _N, fc), jnp.float32),
        mesh=mesh,
        scratch_types=[
            pltpu.VMEM((_TB, _B), jnp.int32),
            pltpu.VMEM((_TB, _B), jnp.int32),
            [pltpu.VMEM((_B, fc), jnp.float32) for _ in range(2)],
            pltpu.VMEM_SHARED((_NPAD, fc), jnp.float32),
            pltpu.SemaphoreType.DMA,
            pltpu.SemaphoreType.DMA,
        ],
        compiler_params=pltpu.CompilerParams(use_tc_tiling_on_sc=False),
    )
    def spmm(y_hbm, srcq_hbm, dst_hbm, out_hbm, src_v, dst_v, rows, acc,
             gsem0, gsem1):
        c = lax.axis_index("c")
        s = lax.axis_index("s")
        gsem = (gsem0, gsem1)

        def g_issue(b, t):
            pltpu.async_copy(y_hbm.at[src_v.at[t]], rows[b], gsem[b])

        def g_wait(b, t):
            pltpu.make_async_copy(y_hbm.at[src_v.at[t]], rows[b],
                                  gsem[b]).wait()
        pltpu.sync_copy(dst_hbm.at[s], dst_v)
        for j in range(cpc):
            q = c * cpc + j
            pltpu.sync_copy(srcq_hbm.at[q, s], src_v)
            pltpu.sync_copy(y_hbm.at[pl.ds(q * _N + s * _RPS, _RPS)],
                            acc.at[pl.ds(s * _RPS, _RPS)])

            @pl.when(s == 0)
            def _():
                pltpu.sync_copy(
                    y_hbm.at[pl.ds(q * _N + _NSUB * _RPS, _RTAIL)],
                    acc.at[pl.ds(_NSUB * _RPS, _RTAIL)])

            plsc.subcore_barrier()

            g_issue(0, 0)

            def rbody(rr, carry):
                r0 = rr * 2
                g_wait(0, r0)
                g_issue(1, r0 + 1)
                pltpu.sync_copy(rows[0], acc.at[dst_v.at[r0]], add=True)
                g_wait(1, r0 + 1)
                g_issue(0, r0 + 2)
                pltpu.sync_copy(rows[1], acc.at[dst_v.at[r0 + 1]], add=True)
                return carry

            lax.fori_loop(0, _TB // 2, rbody, 0)
            g_wait(0, _TB - 1)
            pltpu.sync_copy(rows[0], acc.at[dst_v.at[_TB - 1]], add=True)

            plsc.subcore_barrier()
            pltpu.sync_copy(acc.at[pl.ds(s * _RPS, _RPS)],
                            out_hbm.at[pl.ds(q * _N + s * _RPS, _RPS)])

            @pl.when(s == 0)
            def _():
                pltpu.sync_copy(
                    acc.at[pl.ds(_NSUB * _RPS, _RTAIL)],
                    out_hbm.at[pl.ds(q * _N + _NSUB * _RPS, _RTAIL)])

            if j + 1 < cpc:
                plsc.subcore_barrier()

    return spmm


def _l1_body(x_ref, deg_ref, w_ref, out_ref):
    dinv = lax.rsqrt(deg_ref[...])
    y = jnp.dot(x_ref[...] * dinv, w_ref[...],
                preferred_element_type=jnp.float32)
    for q in range(2):
        out_ref[q] = y[:, q * 64:(q + 1) * 64]


def _make_layer_body(nc_in, nc_out, fco):
    def body(a_ref, deg_ref, b_ref, w_ref, out_ref):
        dinv = lax.rsqrt(deg_ref[...])
        h = jnp.concatenate([a_ref[i] for i in range(nc_in)], axis=1)
        h = jax.nn.relu(h * dinv + b_ref[...])
        y = jnp.dot(h * dinv, w_ref[...], preferred_element_type=jnp.float32)
        for q in range(nc_out):
            out_ref[q] = y[:, q * fco:(q + 1) * fco]
    return body


def _pool_body(a_ref, deg_ref, b_ref, batch_ref, gs_ref, gmp_ref):
    i = pl.program_id(0)
    dinv = lax.rsqrt(deg_ref[...])
    h = jnp.concatenate([a_ref[q] for q in range(8)], axis=1)
    h = jax.nn.relu(h * dinv + b_ref[...])
    gid = lax.broadcasted_iota(jnp.int32, (1, _G), 1)
    onehot = (batch_ref[...] == gid).astype(jnp.float32)
    gs = lax.dot_general(onehot, h, (((0,), (0,)), ((), ())),
                         preferred_element_type=jnp.float32)
    parts = []
    for g in range(_G):
        parts.append(jnp.max(onehot[:, g:g + 1] * h, axis=0, keepdims=True))
    gmp = jnp.concatenate(parts, axis=0)

    @pl.when(i == 0)
    def _():
        gs_ref[...] = gs
        gmp_ref[...] = gmp

    @pl.when(i > 0)
    def _():
        gs_ref[...] += gs
        gmp_ref[...] = jnp.maximum(gmp_ref[...], gmp)


def _mlp_body(batch_ref, gs_ref, gmp_ref, sf_ref,
              Wg1_ref, bg1_ref, Wg2_ref, bg2_ref,
              Ws1_ref, bs1_ref, Ws2_ref, bs2_ref,
              Wf1_ref, bf1_ref, Wf2_ref, bf2_ref, Wo_ref, bo_ref, out_ref):
    gid = lax.broadcasted_iota(jnp.int32, (1, _G), 1)
    onehot = (batch_ref[...] == gid).astype(jnp.float32)
    ones = jnp.ones((_N, 1), jnp.float32)
    counts = lax.dot_general(onehot, ones, (((0,), (0,)), ((), ())),
                             preferred_element_type=jnp.float32)
    gap = gs_ref[...] / jnp.maximum(counts, 1.0)
    comb = jnp.concatenate([gap, gmp_ref[...]], axis=1)
    comb = jax.nn.relu(
        jnp.dot(comb, Wg1_ref[...], preferred_element_type=jnp.float32)
        + bg1_ref[...])
    comb = jax.nn.relu(
        jnp.dot(comb, Wg2_ref[...], preferred_element_type=jnp.float32)
        + bg2_ref[...])
    s = jax.nn.relu(
        jnp.dot(sf_ref[...], Ws1_ref[...], preferred_element_type=jnp.float32)
        + bs1_ref[...])
    s = jax.nn.relu(
        jnp.dot(s, Ws2_ref[...], preferred_element_type=jnp.float32)
        + bs2_ref[...])
    z = jnp.concatenate([comb, s], axis=1)
    z = jax.nn.relu(
        jnp.dot(z, Wf1_ref[...], preferred_element_type=jnp.float32)
        + bf1_ref[...])
    z = jax.nn.relu(
        jnp.dot(z, Wf2_ref[...], preferred_element_type=jnp.float32)
        + bf2_ref[...])
    out_ref[...] = (
        jnp.dot(z, Wo_ref[...], preferred_element_type=jnp.float32)
        + bo_ref[...])


def _layer_call(body, nc_in, fci, nc_out, fco, a, deg2, b, w):
    return pl.pallas_call(
        body,
        grid=(_N // _BN,),
        in_specs=[
            pl.BlockSpec((nc_in, _BN, fci), lambda i: (0, i, 0)),
            pl.BlockSpec((_BN, 1), lambda i: (i, 0)),
            pl.BlockSpec((1, nc_in * fci), lambda i: (0, 0)),
            pl.BlockSpec((nc_in * fci, nc_out * fco), lambda i: (0, 0)),
        ],
        out_specs=pl.BlockSpec((nc_out, _BN, fco), lambda i: (0, i, 0)),
        out_shape=jax.ShapeDtypeStruct((nc_out, _N, fco), jnp.float32),
    )(a, deg2, b, w)


def kernel(x, edge_index, edge_attr, batch, solvent_fingerprint,
           W1, b1, W2, b2, W3, b3, Wg1, bg1, Wg2, bg2,
           Ws1, bs1, Ws2, bs2, Wf1, bf1, Wf2, bf2, Wo, bo):
    src = edge_index[0]
    dst = edge_index[1]
    src_p = jnp.concatenate([src, jnp.zeros((_E2 - _E,), jnp.int32)])
    qoff = (jnp.arange(8, dtype=jnp.int32) * _N)[:, None, None, None]
    srcq = src_p.reshape(1, _NSUB, _TB, _B) + qoff
    dst_p = jnp.concatenate(
        [dst, jnp.full((_E2 - _E,), _N, jnp.int32)]).reshape(_NSUB, _TB, _B)

    spmm16 = _make_spmm(2, 16)
    deg_full = spmm16(jnp.ones((2 * _N, 16), jnp.float32), srcq, dst_p)
    deg2 = deg_full[:_N, :1]
    spmm64a = _make_spmm(2, 64)
    spmm64 = _make_spmm(4, 64)

    y1 = pl.pallas_call(
        _l1_body,
        grid=(_N // _BN,),
        in_specs=[
            pl.BlockSpec((_BN, 128), lambda i: (i, 0)),
            pl.BlockSpec((_BN, 1), lambda i: (i, 0)),
            pl.BlockSpec((128, 128), lambda i: (0, 0)),
        ],
        out_specs=pl.BlockSpec((2, _BN, 64), lambda i: (0, i, 0)),
        out_shape=jax.ShapeDtypeStruct((2, _N, 64), jnp.float32),
    )(x, deg2, W1)
    agg1 = spmm64a(y1.reshape(2 * _N, 64), srcq, dst_p)

    y2 = _layer_call(_make_layer_body(2, 4, 64), 2, 64, 4, 64,
                     agg1.reshape(2, _N, 64), deg2, b1.reshape(1, 128), W2)
    agg2 = spmm64(y2.reshape(4 * _N, 64), srcq, dst_p)

    y3 = _layer_call(_make_layer_body(4, 8, 64), 4, 64, 8, 64,
                     agg2.reshape(4, _N, 64), deg2, b2.reshape(1, 256), W3)
    y3f = y3.reshape(8 * _N, 64)
    agg3 = jnp.concatenate([spmm64(y3f[:4 * _N], srcq, dst_p),
                            spmm64(y3f[4 * _N:], srcq, dst_p)])

    batch2 = batch.reshape(_N, 1)
    gs, gmp = pl.pallas_call(
        _pool_body,
        grid=(_N // _BN,),
        in_specs=[
            pl.BlockSpec((8, _BN, 64), lambda i: (0, i, 0)),
            pl.BlockSpec((_BN, 1), lambda i: (i, 0)),
            pl.BlockSpec((1, 512), lambda i: (0, 0)),
            pl.BlockSpec((_BN, 1), lambda i: (i, 0)),
        ],
        out_specs=[
            pl.BlockSpec((_G, 512), lambda i: (0, 0)),
            pl.BlockSpec((_G, 512), lambda i: (0, 0)),
        ],
        out_shape=[
            jax.ShapeDtypeStruct((_G, 512), jnp.float32),
            jax.ShapeDtypeStruct((_G, 512), jnp.float32),
        ],
    )(agg3.reshape(8, _N, 64), deg2, b3.reshape(1, 512), batch2)

    sf = solvent_fingerprint.reshape(_G, 512)
    out = pl.pallas_call(
        _mlp_body,
        out_shape=jax.ShapeDtypeStruct((_G, 1), jnp.float32),
    )(batch2, gs, gmp, sf,
      Wg1, bg1.reshape(1, -1), Wg2, bg2.reshape(1, -1),
      Ws1, bs1.reshape(1, -1), Ws2, bs2.reshape(1, -1),
      Wf1, bf1.reshape(1, -1), Wf2, bf2.reshape(1, -1), Wo, bo.reshape(1, -1))
    return out


# speedup vs baseline: 2.7744x; 1.1286x over previous
import functools

import jax
import jax.numpy as jnp
from jax import lax
from jax.experimental import pallas as pl
from jax.experimental.pallas import tpu as pltpu
from jax.experimental.pallas import tpu_sc as plsc

_N = 10000
_E = 320000
_G = 64
_NSUB = 16
_B = 128
_TB = 157
_E2 = _NSUB * _TB * _B
_NPAD = 10016
_RPS = 624
_RTAIL = _N - _NSUB * _RPS
_BN = 1000


def _make_spmm(nchunk, fc):
    cpc = nchunk // 2
    mesh = plsc.VectorSubcoreMesh(core_axis_name="c", subcore_axis_name="s")

    @functools.partial(
        pl.kernel,
        out_type=jax.ShapeDtypeStruct((nchunk * _N, fc), jnp.float32),
        mesh=mesh,
        scratch_types=[
            pltpu.VMEM((_TB + 1, _B), jnp.int32),
            pltpu.VMEM((_TB, _B), jnp.int32),
            [pltpu.VMEM((_B, fc), jnp.float32) for _ in range(3)],
            pltpu.VMEM_SHARED((_NPAD, fc), jnp.float32),
            pltpu.SemaphoreType.DMA,
            pltpu.SemaphoreType.DMA,
            pltpu.SemaphoreType.DMA,
        ],
        compiler_params=pltpu.CompilerParams(use_tc_tiling_on_sc=False),
    )
    def spmm(y_hbm, srcq_hbm, dst_hbm, out_hbm, src_v, dst_v, rows, acc,
             gsem0, gsem1, gsem2):
        c = lax.axis_index("c")
        s = lax.axis_index("s")
        gsem = (gsem0, gsem1, gsem2)

        def g_issue(b, t):
            pltpu.async_copy(y_hbm.at[src_v.at[t]], rows[b], gsem[b])

        def g_wait(b, t):
            pltpu.make_async_copy(y_hbm.at[src_v.at[t]], rows[b],
                                  gsem[b]).wait()
        pltpu.sync_copy(dst_hbm.at[s], dst_v)
        for j in range(cpc):
            q = c * cpc + j
            pltpu.sync_copy(srcq_hbm.at[q, s], src_v)
            pltpu.sync_copy(y_hbm.at[pl.ds(q * _N + s * _RPS, _RPS)],
                            acc.at[pl.ds(s * _RPS, _RPS)])

            @pl.when(s == 0)
            def _():
                pltpu.sync_copy(
                    y_hbm.at[pl.ds(q * _N + _NSUB * _RPS, _RTAIL)],
                    acc.at[pl.ds(_NSUB * _RPS, _RTAIL)])

            plsc.subcore_barrier()

            g_issue(0, 0)
            g_issue(1, 1)

            def rbody(rr, carry):
                r0 = rr * 3
                for b in range(3):
                    g_wait(b, r0 + b)
                    g_issue((b + 2) % 3, r0 + b + 2)
                    pltpu.sync_copy(rows[b], acc.at[dst_v.at[r0 + b]],
                                    add=True)
                return carry

            lax.fori_loop(0, _TB // 3, rbody, 0)
            g_wait(0, _TB - 1)
            pltpu.sync_copy(rows[0], acc.at[dst_v.at[_TB - 1]], add=True)
            g_wait(1, _TB)

            plsc.subcore_barrier()
            pltpu.sync_copy(acc.at[pl.ds(s * _RPS, _RPS)],
                            out_hbm.at[pl.ds(q * _N + s * _RPS, _RPS)])

            @pl.when(s == 0)
            def _():
                pltpu.sync_copy(
                    acc.at[pl.ds(_NSUB * _RPS, _RTAIL)],
                    out_hbm.at[pl.ds(q * _N + _NSUB * _RPS, _RTAIL)])

            if j + 1 < cpc:
                plsc.subcore_barrier()

    return spmm


def _l1_body(x_ref, deg_ref, w_ref, out_ref):
    dinv = lax.rsqrt(deg_ref[...])
    y = jnp.dot(x_ref[...] * dinv, w_ref[...],
                preferred_element_type=jnp.float32)
    for q in range(2):
        out_ref[q] = y[:, q * 64:(q + 1) * 64]


def _make_layer_body(nc_in, nc_out, fco):
    def body(a_ref, deg_ref, b_ref, w_ref, out_ref):
        dinv = lax.rsqrt(deg_ref[...])
        h = jnp.concatenate([a_ref[i] for i in range(nc_in)], axis=1)
        h = jax.nn.relu(h * dinv + b_ref[...])
        y = jnp.dot(h * dinv, w_ref[...], preferred_element_type=jnp.float32)
        for q in range(nc_out):
            out_ref[q] = y[:, q * fco:(q + 1) * fco]
    return body


def _pool_body(a_ref, deg_ref, b_ref, batch_ref, gs_ref, gmp_ref):
    i = pl.program_id(0)
    dinv = lax.rsqrt(deg_ref[...])
    h = jnp.concatenate([a_ref[q] for q in range(8)], axis=1)
    h = jax.nn.relu(h * dinv + b_ref[...])
    gid = lax.broadcasted_iota(jnp.int32, (1, _G), 1)
    onehot = (batch_ref[...] == gid).astype(jnp.float32)
    gs = lax.dot_general(onehot, h, (((0,), (0,)), ((), ())),
                         preferred_element_type=jnp.float32)
    parts = []
    for g in range(_G):
        parts.append(jnp.max(onehot[:, g:g + 1] * h, axis=0, keepdims=True))
    gmp = jnp.concatenate(parts, axis=0)

    @pl.when(i == 0)
    def _():
        gs_ref[...] = gs
        gmp_ref[...] = gmp

    @pl.when(i > 0)
    def _():
        gs_ref[...] += gs
        gmp_ref[...] = jnp.maximum(gmp_ref[...], gmp)


def _mlp_body(batch_ref, gs_ref, gmp_ref, sf_ref,
              Wg1_ref, bg1_ref, Wg2_ref, bg2_ref,
              Ws1_ref, bs1_ref, Ws2_ref, bs2_ref,
              Wf1_ref, bf1_ref, Wf2_ref, bf2_ref, Wo_ref, bo_ref, out_ref):
    gid = lax.broadcasted_iota(jnp.int32, (1, _G), 1)
    onehot = (batch_ref[...] == gid).astype(jnp.float32)
    ones = jnp.ones((_N, 1), jnp.float32)
    counts = lax.dot_general(onehot, ones, (((0,), (0,)), ((), ())),
                             preferred_element_type=jnp.float32)
    gap = gs_ref[...] / jnp.maximum(counts, 1.0)
    comb = jnp.concatenate([gap, gmp_ref[...]], axis=1)
    comb = jax.nn.relu(
        jnp.dot(comb, Wg1_ref[...], preferred_element_type=jnp.float32)
        + bg1_ref[...])
    comb = jax.nn.relu(
        jnp.dot(comb, Wg2_ref[...], preferred_element_type=jnp.float32)
        + bg2_ref[...])
    s = jax.nn.relu(
        jnp.dot(sf_ref[...], Ws1_ref[...], preferred_element_type=jnp.float32)
        + bs1_ref[...])
    s = jax.nn.relu(
        jnp.dot(s, Ws2_ref[...], preferred_element_type=jnp.float32)
        + bs2_ref[...])
    z = jnp.concatenate([comb, s], axis=1)
    z = jax.nn.relu(
        jnp.dot(z, Wf1_ref[...], preferred_element_type=jnp.float32)
        + bf1_ref[...])
    z = jax.nn.relu(
        jnp.dot(z, Wf2_ref[...], preferred_element_type=jnp.float32)
        + bf2_ref[...])
    out_ref[...] = (
        jnp.dot(z, Wo_ref[...], preferred_element_type=jnp.float32)
        + bo_ref[...])


def _layer_call(body, nc_in, fci, nc_out, fco, a, deg2, b, w):
    return pl.pallas_call(
        body,
        grid=(_N // _BN,),
        in_specs=[
            pl.BlockSpec((nc_in, _BN, fci), lambda i: (0, i, 0)),
            pl.BlockSpec((_BN, 1), lambda i: (i, 0)),
            pl.BlockSpec((1, nc_in * fci), lambda i: (0, 0)),
            pl.BlockSpec((nc_in * fci, nc_out * fco), lambda i: (0, 0)),
        ],
        out_specs=pl.BlockSpec((nc_out, _BN, fco), lambda i: (0, i, 0)),
        out_shape=jax.ShapeDtypeStruct((nc_out, _N, fco), jnp.float32),
    )(a, deg2, b, w)


def kernel(x, edge_index, edge_attr, batch, solvent_fingerprint,
           W1, b1, W2, b2, W3, b3, Wg1, bg1, Wg2, bg2,
           Ws1, bs1, Ws2, bs2, Wf1, bf1, Wf2, bf2, Wo, bo):
    src = edge_index[0]
    dst = edge_index[1]
    src_p = jnp.concatenate([src, jnp.zeros((_E2 - _E,), jnp.int32)])
    src_r = jnp.concatenate(
        [src_p.reshape(_NSUB, _TB, _B),
         jnp.zeros((_NSUB, 1, _B), jnp.int32)], axis=1)
    qoff = (jnp.arange(8, dtype=jnp.int32) * _N)[:, None, None, None]
    srcq = src_r[None] + qoff
    dst_p = jnp.concatenate(
        [dst, jnp.full((_E2 - _E,), _N, jnp.int32)]).reshape(_NSUB, _TB, _B)

    spmm16 = _make_spmm(2, 16)
    deg_full = spmm16(jnp.ones((2 * _N, 16), jnp.float32), srcq, dst_p)
    deg2 = deg_full[:_N, :1]
    spmm64a = _make_spmm(2, 64)
    spmm64 = _make_spmm(4, 64)

    y1 = pl.pallas_call(
        _l1_body,
        grid=(_N // _BN,),
        in_specs=[
            pl.BlockSpec((_BN, 128), lambda i: (i, 0)),
            pl.BlockSpec((_BN, 1), lambda i: (i, 0)),
            pl.BlockSpec((128, 128), lambda i: (0, 0)),
        ],
        out_specs=pl.BlockSpec((2, _BN, 64), lambda i: (0, i, 0)),
        out_shape=jax.ShapeDtypeStruct((2, _N, 64), jnp.float32),
    )(x, deg2, W1)
    agg1 = spmm64a(y1.reshape(2 * _N, 64), srcq, dst_p)

    y2 = _layer_call(_make_layer_body(2, 4, 64), 2, 64, 4, 64,
                     agg1.reshape(2, _N, 64), deg2, b1.reshape(1, 128), W2)
    agg2 = spmm64(y2.reshape(4 * _N, 64), srcq, dst_p)

    y3 = _layer_call(_make_layer_body(4, 8, 64), 4, 64, 8, 64,
                     agg2.reshape(4, _N, 64), deg2, b2.reshape(1, 256), W3)
    y3f = y3.reshape(8 * _N, 64)
    agg3 = jnp.concatenate([spmm64(y3f[:4 * _N], srcq, dst_p),
                            spmm64(y3f[4 * _N:], srcq, dst_p)])

    batch2 = batch.reshape(_N, 1)
    gs, gmp = pl.pallas_call(
        _pool_body,
        grid=(_N // _BN,),
        in_specs=[
            pl.BlockSpec((8, _BN, 64), lambda i: (0, i, 0)),
            pl.BlockSpec((_BN, 1), lambda i: (i, 0)),
            pl.BlockSpec((1, 512), lambda i: (0, 0)),
            pl.BlockSpec((_BN, 1), lambda i: (i, 0)),
        ],
        out_specs=[
            pl.BlockSpec((_G, 512), lambda i: (0, 0)),
            pl.BlockSpec((_G, 512), lambda i: (0, 0)),
        ],
        out_shape=[
            jax.ShapeDtypeStruct((_G, 512), jnp.float32),
            jax.ShapeDtypeStruct((_G, 512), jnp.float32),
        ],
    )(agg3.reshape(8, _N, 64), deg2, b3.reshape(1, 512), batch2)

    sf = solvent_fingerprint.reshape(_G, 512)
    out = pl.pallas_call(
        _mlp_body,
        out_shape=jax.ShapeDtypeStruct((_G, 1), jnp.float32),
    )(batch2, gs, gmp, sf,
      Wg1, bg1.reshape(1, -1), Wg2, bg2.reshape(1, -1),
      Ws1, bs1.reshape(1, -1), Ws2, bs2.reshape(1, -1),
      Wf1, bf1.reshape(1, -1), Wf2, bf2.reshape(1, -1), Wo, bo.reshape(1, -1))
    return out
